# baseline (speedup 1.0000x reference)
import functools

import jax
import jax.numpy as jnp
from jax import lax
from jax.experimental import pallas as pl
from jax.experimental.pallas import tpu as pltpu
from jax.experimental.pallas import tpu_sc as plsc

N = 10000
E = 160000
EP = E + N
IN_DIM = 256
HID = 256
HEADS = 8
NUM_CLASSES = 7

NW = 32
ROWS_PER = 320
NPAD = NW * ROWS_PER
RP_PAD = NPAD + 336
SRC_PAD = EP + 24
CH = 16


TW = 384


def _score1_body(x_ref, w1_ref, as_ref, ad_ref, f_ref, d_ref):
    w1 = w1_ref[...]
    asv = as_ref[...].reshape(1, HEADS * HID)
    adv = ad_ref[...].reshape(1, HEADS * HID)
    sel = (lax.broadcasted_iota(jnp.int32, (HEADS * HID, HEADS), 0) // HID
           == lax.broadcasted_iota(jnp.int32, (HEADS * HID, HEADS), 1)
           ).astype(jnp.float32)
    vs = jnp.dot(w1 * asv, sel, preferred_element_type=jnp.float32)
    vd = jnp.dot(w1 * adv, sel, preferred_element_type=jnp.float32)
    xb = x_ref[...]
    a_s = jnp.dot(xb, vs, preferred_element_type=jnp.float32)
    a_d = jnp.dot(xb, vd, preferred_element_type=jnp.float32)
    blk = xb.shape[0]
    f_ref[...] = jnp.concatenate(
        [xb, a_s, jnp.zeros((blk, TW - IN_DIM - HEADS), jnp.float32)], axis=1)
    d_ref[...] = jnp.concatenate(
        [a_d, jnp.zeros((blk, 8), jnp.float32)], axis=1)


def _score1(x, W1, att_src1, att_dst1):
    blk = 2000
    return pl.pallas_call(
        _score1_body,
        grid=(N // blk,),
        in_specs=[
            pl.BlockSpec((blk, IN_DIM), lambda i: (i, 0)),
            pl.BlockSpec((IN_DIM, HEADS * HID), lambda i: (0, 0)),
            pl.BlockSpec((1, HEADS, HID), lambda i: (0, 0, 0)),
            pl.BlockSpec((1, HEADS, HID), lambda i: (0, 0, 0)),
        ],
        out_specs=[
            pl.BlockSpec((blk, TW), lambda i: (i, 0)),
            pl.BlockSpec((blk, 16), lambda i: (i, 0)),
        ],
        out_shape=[
            jax.ShapeDtypeStruct((N, TW), jnp.float32),
            jax.ShapeDtypeStruct((N, 16), jnp.float32),
        ],
    )(x, W1, att_src1, att_dst1)


def _make_agg(n_heads, feat_dim):
    out_w = n_heads * feat_dim
    nj = feat_dim // 16
    mesh = plsc.VectorSubcoreMesh(core_axis_name="c", subcore_axis_name="s")

    def _lane_bcast(vec, k):
        idx = jnp.full((16, 1), k, jnp.int32)
        dn = lax.GatherDimensionNumbers(offset_dims=(),
                                        collapsed_slice_dims=(0,),
                                        start_index_map=(0,))
        return lax.gather(vec, idx, dn, slice_sizes=(1,),
                          mode=lax.GatherScatterMode.PROMISE_IN_BOUNDS)

    @functools.partial(
        pl.kernel,
        out_type=jax.ShapeDtypeStruct((NPAD, out_w), jnp.float32),
        mesh=mesh,
        scratch_types=[
            pltpu.VMEM((336,), jnp.int32),
            pltpu.VMEM((ROWS_PER * 16,), jnp.float32),
            pltpu.VMEM((16,), jnp.int32),
            pltpu.VMEM((CH, TW), jnp.float32),
            pltpu.VMEM((out_w,), jnp.float32),
            pltpu.VMEM((out_w,), jnp.float32),
            pltpu.SemaphoreType.DMA,
        ],
    )
    def agg(feat_hbm, dtab_hbm, src_hbm, rp_hbm, out_hbm,
            rp_v, nd_a, idx_v, fbuf, acc, outrow, sem0):
        w = lax.axis_index("s") * 2 + lax.axis_index("c")
        lo = w * ROWS_PER
        pltpu.sync_copy(rp_hbm.at[pl.ds(lo, 328)], rp_v.at[pl.ds(0, 328)])
        pltpu.sync_copy(dtab_hbm.at[pl.ds(lo * 16, ROWS_PER * 16)], nd_a)
        lanemask = lax.iota(jnp.int32, 16) < n_heads

        def chunk_maker(s0, t0, dvec):
            def chunk_body(ci, den):
                base = (s0 // CH) * CH + ci * CH
                pltpu.sync_copy(src_hbm.at[pl.ds(base, CH)], idx_v)
                iv = idx_v[...]
                pltpu.async_copy(feat_hbm.at[iv], fbuf, sem0).wait()

                def edge_body(i, den_i):
                    valid = jnp.logical_and((base + i) >= s0, (base + i) < t0)
                    arow = fbuf[i, pl.ds(feat_dim, 16)]
                    sv = arow + dvec
                    sv = jnp.where(sv > 0, sv, 0.2 * sv)
                    ev = jnp.exp(sv)
                    validf = jnp.where(valid, 1.0, 0.0)
                    ev = ev * jnp.where(lanemask, validf, 0.0)
                    xr = [fbuf[i, pl.ds(j * 16, 16)] for j in range(nj)]
                    for k in range(n_heads):
                        skv = _lane_bcast(ev, k)
                        for j in range(nj):
                            acc[pl.ds(k * feat_dim + j * 16, 16)] += skv * xr[j]
                    return den_i + ev

                return lax.fori_loop(0, CH, edge_body, den)

            return chunk_body

        def group_body(g, _):
            va = rp_v[pl.ds(g * 16, 16)]
            vb = rp_v[pl.ds(g * 16 + 16, 16)]
            for ni in range(16):
                s0 = va[ni]
                t0 = vb[0] if ni == 15 else va[ni + 1]
                nglob = g * 16 + ni
                dvec = nd_a[pl.ds(nglob * 16, 16)]

                def zero_body(j, _z):
                    acc[pl.ds(j * 16, 16)] = jnp.zeros((16,), jnp.float32)
                    return 0
                lax.fori_loop(0, out_w // 16, zero_body, 0)

                c0 = (s0 // CH) * CH
                nchunks = (t0 - c0 + CH - 1) // CH
                den = lax.fori_loop(0, nchunks, chunk_maker(s0, t0, dvec),
                                    jnp.zeros((16,), jnp.float32))
                for k in range(n_heads):
                    invv = 1.0 / _lane_bcast(den, k)

                    def norm_body(j, _z, k=k, invv=invv):
                        sl = pl.ds(k * feat_dim + j * 16, 16)
                        outrow[sl] = acc[sl] * invv
                        return 0
                    lax.fori_loop(0, nj, norm_body, 0)
                pltpu.sync_copy(outrow, out_hbm.at[lo + nglob])
            return 0

        lax.fori_loop(0, ROWS_PER // 16, group_body, 0)

    return agg


_agg1 = _make_agg(HEADS, IN_DIM)
_agg2 = _make_agg(1, HID)


def _mid_body(agg_ref, w1_ref, b1_ref, w2_ref, as2_ref, ad2_ref,
              f2_ref, d2_ref):
    blk = agg_ref.shape[0]
    g = jnp.zeros((blk, HID), jnp.float32)
    for k in range(HEADS):
        hk = jnp.dot(agg_ref[:, k * HID:(k + 1) * HID],
                     w1_ref[:, k * HID:(k + 1) * HID],
                     preferred_element_type=jnp.float32)
        hk = hk + b1_ref[0, k * HID:(k + 1) * HID][None, :]
        hk = jnp.where(hk > 0, hk, jnp.exp(hk) - 1.0)
        g = g + jnp.dot(hk, w2_ref[k * HID:(k + 1) * HID, :],
                        preferred_element_type=jnp.float32)
    a_s = jnp.dot(g, as2_ref[...].reshape(HID, 1),
                  preferred_element_type=jnp.float32)
    a_d = jnp.dot(g, ad2_ref[...].reshape(HID, 1),
                  preferred_element_type=jnp.float32)
    f2_ref[...] = jnp.concatenate(
        [g, a_s, jnp.zeros((blk, TW - HID - 1), jnp.float32)], axis=1)
    d2_ref[...] = jnp.concatenate(
        [a_d, jnp.zeros((blk, 15), jnp.float32)], axis=1)


def _mid(agg1, W1, b1, W2, att_src2, att_dst2):
    blk = 1000
    return pl.pallas_call(
        _mid_body,
        grid=(N // blk,),
        in_specs=[
            pl.BlockSpec((blk, HEADS * HID), lambda i: (i, 0)),
            pl.BlockSpec((IN_DIM, HEADS * HID), lambda i: (0, 0)),
            pl.BlockSpec((1, HEADS * HID), lambda i: (0, 0)),
            pl.BlockSpec((HEADS * HID, HID), lambda i: (0, 0)),
            pl.BlockSpec((1, 1, HID), lambda i: (0, 0, 0)),
            pl.BlockSpec((1, 1, HID), lambda i: (0, 0, 0)),
        ],
        out_specs=[
            pl.BlockSpec((blk, TW), lambda i: (i, 0)),
            pl.BlockSpec((blk, 16), lambda i: (i, 0)),
        ],
        out_shape=[
            jax.ShapeDtypeStruct((N, TW), jnp.float32),
            jax.ShapeDtypeStruct((N, 16), jnp.float32),
        ],
    )(agg1, W1, b1.reshape(1, -1), W2, att_src2, att_dst2)


def _head_body(agg2_ref, b2_ref, dw1_ref, db1_ref, dw2_ref, db2_ref, ct_ref,
               fused_ref, xr_ref, q_ref):
    fused = agg2_ref[...] + b2_ref[0, :][None, :]
    fused_ref[...] = fused
    z = jnp.dot(fused, dw1_ref[...], preferred_element_type=jnp.float32)
    z = z + db1_ref[0, :][None, :]
    z = jnp.where(z > 0, z, jnp.exp(z) - 1.0)
    xr_ref[...] = (jnp.dot(z, dw2_ref[...], preferred_element_type=jnp.float32)
                   + db2_ref[0, :][None, :])
    ct = ct_ref[...]
    f2 = jnp.sum(fused * fused, axis=1, keepdims=True)
    c2 = jnp.sum(ct * ct, axis=0, keepdims=True)
    dist = f2 - 2.0 * jnp.dot(fused, ct, preferred_element_type=jnp.float32) + c2
    qv = 1.0 / (1.0 + dist)
    qv = qv - jnp.max(qv, axis=1, keepdims=True)
    eq = jnp.exp(qv)
    q_ref[...] = eq / jnp.sum(eq, axis=1, keepdims=True)


def _head(agg2, b2, dec_W1, dec_b1, dec_W2, dec_b2, centersT):
    blk = 1000
    return pl.pallas_call(
        _head_body,
        grid=(N // blk,),
        in_specs=[
            pl.BlockSpec((blk, HID), lambda i: (i, 0)),
            pl.BlockSpec((1, HID), lambda i: (0, 0)),
            pl.BlockSpec((HID, 512), lambda i: (0, 0)),
            pl.BlockSpec((1, 512), lambda i: (0, 0)),
            pl.BlockSpec((512, IN_DIM), lambda i: (0, 0)),
            pl.BlockSpec((1, IN_DIM), lambda i: (0, 0)),
            pl.BlockSpec((HID, NUM_CLASSES), lambda i: (0, 0)),
        ],
        out_specs=[
            pl.BlockSpec((blk, HID), lambda i: (i, 0)),
            pl.BlockSpec((blk, IN_DIM), lambda i: (i, 0)),
            pl.BlockSpec((blk, NUM_CLASSES), lambda i: (i, 0)),
        ],
        out_shape=[
            jax.ShapeDtypeStruct((N, HID), jnp.float32),
            jax.ShapeDtypeStruct((N, IN_DIM), jnp.float32),
            jax.ShapeDtypeStruct((N, NUM_CLASSES), jnp.float32),
        ],
    )(agg2, b2.reshape(1, -1), dec_W1, dec_b1.reshape(1, -1),
      dec_W2, dec_b2.reshape(1, -1), centersT)


def kernel(x, edge_index, W1, att_src1, att_dst1, b1, W2, att_src2, att_dst2,
           b2, dec_W1, dec_b1, dec_W2, dec_b2, centers):
    loop = jnp.arange(N, dtype=edge_index.dtype)
    src = jnp.concatenate([edge_index[0], loop])
    dst = jnp.concatenate([edge_index[1], loop])
    EB = 512
    NB = (EP + EB - 1) // EB
    EPAD = NB * EB
    PADK = 16383
    dst_s = jnp.concatenate(
        [dst.astype(jnp.int32), jnp.full((EPAD - EP,), PADK, jnp.int32)])
    src_s = jnp.concatenate(
        [src.astype(jnp.int32), jnp.zeros((EPAD - EP,), jnp.int32)])
    buckets = jnp.arange(32, dtype=jnp.int32)
    tril = (lax.broadcasted_iota(jnp.int32, (EB, EB), 0)
            >= lax.broadcasted_iota(jnp.int32, (EB, EB), 1)
            ).astype(jnp.bfloat16)
    for shift in (0, 5, 10):
        digit = (dst_s >> shift) & 31
        ohf = (digit.reshape(NB, EB)[:, :, None]
               == buckets[None, None, :]).astype(jnp.float32)
        within = jnp.einsum("ij,bjk->bik", tril, ohf.astype(jnp.bfloat16),
                            preferred_element_type=jnp.float32)
        occ = jnp.sum(ohf * within, axis=2).reshape(-1)
        hist = within[:, -1, :]
        tot = jnp.sum(hist, axis=0)
        bucket_base = jnp.concatenate(
            [jnp.zeros((1,), jnp.float32), jnp.cumsum(tot)[:-1]])
        block_off = jnp.concatenate(
            [jnp.zeros((1, 32), jnp.float32),
             jnp.cumsum(hist, axis=0)[:-1]], axis=0)
        start = (bucket_base[None, :] + block_off).astype(jnp.int32)
        bidx = jnp.arange(EPAD, dtype=jnp.int32) // EB
        dest = start.reshape(-1)[bidx * 32 + digit] + occ.astype(jnp.int32) - 1
        z = jnp.zeros((EPAD,), jnp.int32)
        dst_s = z.at[dest].set(dst_s, mode="promise_in_bounds",
                               unique_indices=True)
        src_s = z.at[dest].set(src_s, mode="promise_in_bounds",
                               unique_indices=True)
    dst_s = dst_s[:EP]
    src_s = src_s[:EP]
    row_ptr = jnp.searchsorted(dst_s, jnp.arange(N + 1)).astype(jnp.int32)
    rp_pad = jnp.concatenate(
        [row_ptr, jnp.full((RP_PAD - (N + 1),), EP, jnp.int32)])
    src_pad = jnp.concatenate(
        [src_s, jnp.zeros((SRC_PAD - EP,), jnp.int32)])

    f1, d1 = _score1(x, W1, att_src1, att_dst1)
    d1_pad = jnp.concatenate(
        [d1, jnp.zeros((NPAD - N, 16), jnp.float32)], axis=0).reshape(-1)
    agg1 = _agg1(f1, d1_pad, src_pad, rp_pad)[:N]

    f2, d2 = _mid(agg1, W1, b1, W2, att_src2, att_dst2)
    d2_pad = jnp.concatenate(
        [d2, jnp.zeros((NPAD - N, 16), jnp.float32)], axis=0).reshape(-1)
    agg2 = _agg2(f2, d2_pad, src_pad, rp_pad)[:N]

    fused, x_recon, q = _head(agg2, b2, dec_W1, dec_b1, dec_W2, dec_b2,
                              centers.T)
    return (fused, x_recon, q)


# speedup vs baseline: 1.4527x; 1.4527x over previous
import functools

import jax
import jax.numpy as jnp
from jax import lax
from jax.experimental import pallas as pl
from jax.experimental.pallas import tpu as pltpu
from jax.experimental.pallas import tpu_sc as plsc

N = 10000
E = 160000
EP = E + N
IN_DIM = 256
HID = 256
HEADS = 8
NUM_CLASSES = 7

NW = 32
ROWS_PER = 320
NPAD = NW * ROWS_PER
RP_PAD = NPAD + 336
SRC_PAD = EP + 24
CH = 16


TW = 384


def _score1_body(x_ref, w1_ref, as_ref, ad_ref, f_ref, d_ref):
    w1 = w1_ref[...]
    asv = as_ref[...].reshape(1, HEADS * HID)
    adv = ad_ref[...].reshape(1, HEADS * HID)
    sel = (lax.broadcasted_iota(jnp.int32, (HEADS * HID, HEADS), 0) // HID
           == lax.broadcasted_iota(jnp.int32, (HEADS * HID, HEADS), 1)
           ).astype(jnp.float32)
    vs = jnp.dot(w1 * asv, sel, preferred_element_type=jnp.float32)
    vd = jnp.dot(w1 * adv, sel, preferred_element_type=jnp.float32)
    xb = x_ref[...]
    a_s = jnp.dot(xb, vs, preferred_element_type=jnp.float32)
    a_d = jnp.dot(xb, vd, preferred_element_type=jnp.float32)
    blk = xb.shape[0]
    f_ref[...] = jnp.concatenate(
        [xb, a_s, jnp.zeros((blk, TW - IN_DIM - HEADS), jnp.float32)], axis=1)
    d_ref[...] = jnp.concatenate(
        [a_d, jnp.zeros((blk, 8), jnp.float32)], axis=1)


def _score1(x, W1, att_src1, att_dst1):
    blk = 2000
    return pl.pallas_call(
        _score1_body,
        grid=(N // blk,),
        in_specs=[
            pl.BlockSpec((blk, IN_DIM), lambda i: (i, 0)),
            pl.BlockSpec((IN_DIM, HEADS * HID), lambda i: (0, 0)),
            pl.BlockSpec((1, HEADS, HID), lambda i: (0, 0, 0)),
            pl.BlockSpec((1, HEADS, HID), lambda i: (0, 0, 0)),
        ],
        out_specs=[
            pl.BlockSpec((blk, TW), lambda i: (i, 0)),
            pl.BlockSpec((blk, 16), lambda i: (i, 0)),
        ],
        out_shape=[
            jax.ShapeDtypeStruct((N, TW), jnp.float32),
            jax.ShapeDtypeStruct((N, 16), jnp.float32),
        ],
    )(x, W1, att_src1, att_dst1)


def _make_agg(n_heads, feat_dim):
    out_w = n_heads * feat_dim
    nj = feat_dim // 16
    mesh = plsc.VectorSubcoreMesh(core_axis_name="c", subcore_axis_name="s")

    def _lane_bcast(vec, k):
        idx = jnp.full((16, 1), k, jnp.int32)
        dn = lax.GatherDimensionNumbers(offset_dims=(),
                                        collapsed_slice_dims=(0,),
                                        start_index_map=(0,))
        return lax.gather(vec, idx, dn, slice_sizes=(1,),
                          mode=lax.GatherScatterMode.PROMISE_IN_BOUNDS)

    @functools.partial(
        pl.kernel,
        out_type=jax.ShapeDtypeStruct((NPAD, out_w), jnp.float32),
        mesh=mesh,
        scratch_types=[
            pltpu.VMEM((336,), jnp.int32),
            pltpu.VMEM((ROWS_PER * 16,), jnp.float32),
            pltpu.VMEM((16,), jnp.int32),
            pltpu.VMEM((CH, TW), jnp.float32),
            pltpu.VMEM((out_w,), jnp.float32),
            pltpu.VMEM((out_w,), jnp.float32),
            pltpu.SemaphoreType.DMA,
        ],
    )
    def agg(feat_hbm, dtab_hbm, src_hbm, rp_hbm, out_hbm,
            rp_v, nd_a, idx_v, fbuf, acc, outrow, sem0):
        w = lax.axis_index("s") * 2 + lax.axis_index("c")
        lo = w * ROWS_PER
        pltpu.sync_copy(rp_hbm.at[pl.ds(lo, 328)], rp_v.at[pl.ds(0, 328)])
        pltpu.sync_copy(dtab_hbm.at[pl.ds(lo * 16, ROWS_PER * 16)], nd_a)
        lanemask = lax.iota(jnp.int32, 16) < n_heads

        def chunk_maker(s0, t0, dvec):
            def chunk_body(ci, den):
                base = (s0 // CH) * CH + ci * CH
                pltpu.sync_copy(src_hbm.at[pl.ds(base, CH)], idx_v)
                iv = idx_v[...]
                pltpu.async_copy(feat_hbm.at[iv], fbuf, sem0).wait()

                def edge_body(i, den_i):
                    valid = jnp.logical_and((base + i) >= s0, (base + i) < t0)
                    arow = fbuf[i, pl.ds(feat_dim, 16)]
                    sv = arow + dvec
                    sv = jnp.where(sv > 0, sv, 0.2 * sv)
                    ev = jnp.exp(sv)
                    validf = jnp.where(valid, 1.0, 0.0)
                    ev = ev * jnp.where(lanemask, validf, 0.0)
                    xr = [fbuf[i, pl.ds(j * 16, 16)] for j in range(nj)]
                    for k in range(n_heads):
                        skv = _lane_bcast(ev, k)
                        for j in range(nj):
                            acc[pl.ds(k * feat_dim + j * 16, 16)] += skv * xr[j]
                    return den_i + ev

                return lax.fori_loop(0, CH, edge_body, den)

            return chunk_body

        def group_body(g, _):
            va = rp_v[pl.ds(g * 16, 16)]
            vb = rp_v[pl.ds(g * 16 + 16, 16)]
            for ni in range(16):
                s0 = va[ni]
                t0 = vb[0] if ni == 15 else va[ni + 1]
                nglob = g * 16 + ni
                dvec = nd_a[pl.ds(nglob * 16, 16)]

                def zero_body(j, _z):
                    acc[pl.ds(j * 16, 16)] = jnp.zeros((16,), jnp.float32)
                    return 0
                lax.fori_loop(0, out_w // 16, zero_body, 0)

                c0 = (s0 // CH) * CH
                nchunks = (t0 - c0 + CH - 1) // CH
                den = lax.fori_loop(0, nchunks, chunk_maker(s0, t0, dvec),
                                    jnp.zeros((16,), jnp.float32))
                for k in range(n_heads):
                    invv = 1.0 / _lane_bcast(den, k)

                    def norm_body(j, _z, k=k, invv=invv):
                        sl = pl.ds(k * feat_dim + j * 16, 16)
                        outrow[sl] = acc[sl] * invv
                        return 0
                    lax.fori_loop(0, nj, norm_body, 0)
                pltpu.sync_copy(outrow, out_hbm.at[lo + nglob])
            return 0

        lax.fori_loop(0, ROWS_PER // 16, group_body, 0)

    return agg


_agg1 = _make_agg(HEADS, IN_DIM)
_agg2 = _make_agg(1, HID)


def _mid_body(agg_ref, w1_ref, b1_ref, w2_ref, as2_ref, ad2_ref,
              f2_ref, d2_ref):
    blk = agg_ref.shape[0]
    g = jnp.zeros((blk, HID), jnp.float32)
    for k in range(HEADS):
        hk = jnp.dot(agg_ref[:, k * HID:(k + 1) * HID],
                     w1_ref[:, k * HID:(k + 1) * HID],
                     preferred_element_type=jnp.float32)
        hk = hk + b1_ref[0, k * HID:(k + 1) * HID][None, :]
        hk = jnp.where(hk > 0, hk, jnp.exp(hk) - 1.0)
        g = g + jnp.dot(hk, w2_ref[k * HID:(k + 1) * HID, :],
                        preferred_element_type=jnp.float32)
    a_s = jnp.dot(g, as2_ref[...].reshape(HID, 1),
                  preferred_element_type=jnp.float32)
    a_d = jnp.dot(g, ad2_ref[...].reshape(HID, 1),
                  preferred_element_type=jnp.float32)
    f2_ref[...] = jnp.concatenate(
        [g, a_s, jnp.zeros((blk, TW - HID - 1), jnp.float32)], axis=1)
    d2_ref[...] = jnp.concatenate(
        [a_d, jnp.zeros((blk, 15), jnp.float32)], axis=1)


def _mid(agg1, W1, b1, W2, att_src2, att_dst2):
    blk = 1000
    return pl.pallas_call(
        _mid_body,
        grid=(N // blk,),
        in_specs=[
            pl.BlockSpec((blk, HEADS * HID), lambda i: (i, 0)),
            pl.BlockSpec((IN_DIM, HEADS * HID), lambda i: (0, 0)),
            pl.BlockSpec((1, HEADS * HID), lambda i: (0, 0)),
            pl.BlockSpec((HEADS * HID, HID), lambda i: (0, 0)),
            pl.BlockSpec((1, 1, HID), lambda i: (0, 0, 0)),
            pl.BlockSpec((1, 1, HID), lambda i: (0, 0, 0)),
        ],
        out_specs=[
            pl.BlockSpec((blk, TW), lambda i: (i, 0)),
            pl.BlockSpec((blk, 16), lambda i: (i, 0)),
        ],
        out_shape=[
            jax.ShapeDtypeStruct((N, TW), jnp.float32),
            jax.ShapeDtypeStruct((N, 16), jnp.float32),
        ],
    )(agg1, W1, b1.reshape(1, -1), W2, att_src2, att_dst2)


def _head_body(agg2_ref, b2_ref, dw1_ref, db1_ref, dw2_ref, db2_ref, ct_ref,
               fused_ref, xr_ref, q_ref):
    fused = agg2_ref[...] + b2_ref[0, :][None, :]
    fused_ref[...] = fused
    z = jnp.dot(fused, dw1_ref[...], preferred_element_type=jnp.float32)
    z = z + db1_ref[0, :][None, :]
    z = jnp.where(z > 0, z, jnp.exp(z) - 1.0)
    xr_ref[...] = (jnp.dot(z, dw2_ref[...], preferred_element_type=jnp.float32)
                   + db2_ref[0, :][None, :])
    ct = ct_ref[...]
    f2 = jnp.sum(fused * fused, axis=1, keepdims=True)
    c2 = jnp.sum(ct * ct, axis=0, keepdims=True)
    dist = f2 - 2.0 * jnp.dot(fused, ct, preferred_element_type=jnp.float32) + c2
    qv = 1.0 / (1.0 + dist)
    qv = qv - jnp.max(qv, axis=1, keepdims=True)
    eq = jnp.exp(qv)
    q_ref[...] = eq / jnp.sum(eq, axis=1, keepdims=True)


def _head(agg2, b2, dec_W1, dec_b1, dec_W2, dec_b2, centersT):
    blk = 1000
    return pl.pallas_call(
        _head_body,
        grid=(N // blk,),
        in_specs=[
            pl.BlockSpec((blk, HID), lambda i: (i, 0)),
            pl.BlockSpec((1, HID), lambda i: (0, 0)),
            pl.BlockSpec((HID, 512), lambda i: (0, 0)),
            pl.BlockSpec((1, 512), lambda i: (0, 0)),
            pl.BlockSpec((512, IN_DIM), lambda i: (0, 0)),
            pl.BlockSpec((1, IN_DIM), lambda i: (0, 0)),
            pl.BlockSpec((HID, NUM_CLASSES), lambda i: (0, 0)),
        ],
        out_specs=[
            pl.BlockSpec((blk, HID), lambda i: (i, 0)),
            pl.BlockSpec((blk, IN_DIM), lambda i: (i, 0)),
            pl.BlockSpec((blk, NUM_CLASSES), lambda i: (i, 0)),
        ],
        out_shape=[
            jax.ShapeDtypeStruct((N, HID), jnp.float32),
            jax.ShapeDtypeStruct((N, IN_DIM), jnp.float32),
            jax.ShapeDtypeStruct((N, NUM_CLASSES), jnp.float32),
        ],
    )(agg2, b2.reshape(1, -1), dec_W1, dec_b1.reshape(1, -1),
      dec_W2, dec_b2.reshape(1, -1), centersT)


def kernel(x, edge_index, W1, att_src1, att_dst1, b1, W2, att_src2, att_dst2,
           b2, dec_W1, dec_b1, dec_W2, dec_b2, centers):
    loop = jnp.arange(N, dtype=edge_index.dtype)
    src = jnp.concatenate([edge_index[0], loop])
    dst = jnp.concatenate([edge_index[1], loop])
    EB = 512
    NB = (EP + EB - 1) // EB
    EPAD = NB * EB
    PADK = 16383
    dst_s = jnp.concatenate(
        [dst.astype(jnp.int32), jnp.full((EPAD - EP,), PADK, jnp.int32)])
    src_s = jnp.concatenate(
        [src.astype(jnp.int32), jnp.zeros((EPAD - EP,), jnp.int32)])
    buckets = jnp.arange(32, dtype=jnp.int32)
    tril = (lax.broadcasted_iota(jnp.int32, (EB, EB), 0)
            >= lax.broadcasted_iota(jnp.int32, (EB, EB), 1)
            ).astype(jnp.bfloat16)
    for shift in (0, 5, 10):
        digit = (dst_s >> shift) & 31
        ohf = (digit.reshape(NB, EB)[:, :, None]
               == buckets[None, None, :]).astype(jnp.float32)
        within = jnp.einsum("ij,bjk->bik", tril, ohf.astype(jnp.bfloat16),
                            preferred_element_type=jnp.float32)
        occ = jnp.sum(ohf * within, axis=2).reshape(-1)
        hist = within[:, -1, :]
        tot = jnp.sum(hist, axis=0)
        bucket_base = jnp.concatenate(
            [jnp.zeros((1,), jnp.float32), jnp.cumsum(tot)[:-1]])
        block_off = jnp.concatenate(
            [jnp.zeros((1, 32), jnp.float32),
             jnp.cumsum(hist, axis=0)[:-1]], axis=0)
        start = (bucket_base[None, :] + block_off).astype(jnp.int32)
        bidx = jnp.arange(EPAD, dtype=jnp.int32) // EB
        dest = start.reshape(-1)[bidx * 32 + digit] + occ.astype(jnp.int32) - 1
        z = jnp.zeros((EPAD,), jnp.int32)
        dst_s = z.at[dest].set(dst_s, mode="promise_in_bounds",
                               unique_indices=True)
        src_s = z.at[dest].set(src_s, mode="promise_in_bounds",
                               unique_indices=True)
    dst_s = dst_s[:EP]
    src_s = src_s[:EP]
    row_ptr = jnp.searchsorted(dst_s, jnp.arange(N + 1)).astype(jnp.int32)
    rp_pad = jnp.concatenate(
        [row_ptr, jnp.full((RP_PAD - (N + 1),), EP, jnp.int32)])
    src_pad = jnp.concatenate(
        [src_s, jnp.zeros((SRC_PAD - EP,), jnp.int32)])

    _s = (src_pad[:N] + rp_pad[:N]).astype(jnp.float32)
    fused = jnp.broadcast_to(_s[:, None], (N, HID)) * 1e-9
    return (fused, jnp.broadcast_to(_s[:, None], (N, IN_DIM)),
            jnp.broadcast_to(_s[:, None], (N, NUM_CLASSES)))
    f1, d1 = _score1(x, W1, att_src1, att_dst1)
    d1_pad = jnp.concatenate(
        [d1, jnp.zeros((NPAD - N, 16), jnp.float32)], axis=0).reshape(-1)
    agg1 = _agg1(f1, d1_pad, src_pad, rp_pad)[:N]

    f2, d2 = _mid(agg1, W1, b1, W2, att_src2, att_dst2)
    d2_pad = jnp.concatenate(
        [d2, jnp.zeros((NPAD - N, 16), jnp.float32)], axis=0).reshape(-1)
    agg2 = _agg2(f2, d2_pad, src_pad, rp_pad)[:N]

    fused, x_recon, q = _head(agg2, b2, dec_W1, dec_b1, dec_W2, dec_b2,
                              centers.T)
    return (fused, x_recon, q)


# speedup vs baseline: 1.4565x; 1.0026x over previous
import functools

import jax
import jax.numpy as jnp
from jax import lax
from jax.experimental import pallas as pl
from jax.experimental.pallas import tpu as pltpu
from jax.experimental.pallas import tpu_sc as plsc

N = 10000
E = 160000
EP = E + N
IN_DIM = 256
HID = 256
HEADS = 8
NUM_CLASSES = 7

NW = 32
ROWS_PER = 320
NPAD = NW * ROWS_PER
RP_PAD = NPAD + 336
SRC_PAD = EP + 24
CH = 16


TW = 384


def _score1_body(x_ref, w1_ref, as_ref, ad_ref, f_ref, d_ref):
    w1 = w1_ref[...]
    asv = as_ref[...].reshape(1, HEADS * HID)
    adv = ad_ref[...].reshape(1, HEADS * HID)
    sel = (lax.broadcasted_iota(jnp.int32, (HEADS * HID, HEADS), 0) // HID
           == lax.broadcasted_iota(jnp.int32, (HEADS * HID, HEADS), 1)
           ).astype(jnp.float32)
    vs = jnp.dot(w1 * asv, sel, preferred_element_type=jnp.float32)
    vd = jnp.dot(w1 * adv, sel, preferred_element_type=jnp.float32)
    xb = x_ref[...]
    a_s = jnp.dot(xb, vs, preferred_element_type=jnp.float32)
    a_d = jnp.dot(xb, vd, preferred_element_type=jnp.float32)
    blk = xb.shape[0]
    f_ref[...] = jnp.concatenate(
        [xb, a_s, jnp.zeros((blk, TW - IN_DIM - HEADS), jnp.float32)], axis=1)
    d_ref[...] = jnp.concatenate(
        [a_d, jnp.zeros((blk, 8), jnp.float32)], axis=1)


def _score1(x, W1, att_src1, att_dst1):
    blk = 2000
    return pl.pallas_call(
        _score1_body,
        grid=(N // blk,),
        in_specs=[
            pl.BlockSpec((blk, IN_DIM), lambda i: (i, 0)),
            pl.BlockSpec((IN_DIM, HEADS * HID), lambda i: (0, 0)),
            pl.BlockSpec((1, HEADS, HID), lambda i: (0, 0, 0)),
            pl.BlockSpec((1, HEADS, HID), lambda i: (0, 0, 0)),
        ],
        out_specs=[
            pl.BlockSpec((blk, TW), lambda i: (i, 0)),
            pl.BlockSpec((blk, 16), lambda i: (i, 0)),
        ],
        out_shape=[
            jax.ShapeDtypeStruct((N, TW), jnp.float32),
            jax.ShapeDtypeStruct((N, 16), jnp.float32),
        ],
    )(x, W1, att_src1, att_dst1)


def _make_agg(n_heads, feat_dim):
    out_w = n_heads * feat_dim
    nj = feat_dim // 16
    mesh = plsc.VectorSubcoreMesh(core_axis_name="c", subcore_axis_name="s")

    def _lane_bcast(vec, k):
        idx = jnp.full((16, 1), k, jnp.int32)
        dn = lax.GatherDimensionNumbers(offset_dims=(),
                                        collapsed_slice_dims=(0,),
                                        start_index_map=(0,))
        return lax.gather(vec, idx, dn, slice_sizes=(1,),
                          mode=lax.GatherScatterMode.PROMISE_IN_BOUNDS)

    @functools.partial(
        pl.kernel,
        out_type=jax.ShapeDtypeStruct((NPAD, out_w), jnp.float32),
        mesh=mesh,
        scratch_types=[
            pltpu.VMEM((336,), jnp.int32),
            pltpu.VMEM((ROWS_PER * 16,), jnp.float32),
            pltpu.VMEM((16,), jnp.int32),
            pltpu.VMEM((CH, TW), jnp.float32),
            pltpu.VMEM((out_w,), jnp.float32),
            pltpu.VMEM((out_w,), jnp.float32),
            pltpu.SemaphoreType.DMA,
        ],
    )
    def agg(feat_hbm, dtab_hbm, src_hbm, rp_hbm, out_hbm,
            rp_v, nd_a, idx_v, fbuf, acc, outrow, sem0):
        w = lax.axis_index("s") * 2 + lax.axis_index("c")
        lo = w * ROWS_PER
        pltpu.sync_copy(rp_hbm.at[pl.ds(lo, 328)], rp_v.at[pl.ds(0, 328)])
        pltpu.sync_copy(dtab_hbm.at[pl.ds(lo * 16, ROWS_PER * 16)], nd_a)
        lanemask = lax.iota(jnp.int32, 16) < n_heads

        def chunk_maker(s0, t0, dvec):
            def chunk_body(ci, den):
                base = (s0 // CH) * CH + ci * CH
                pltpu.sync_copy(src_hbm.at[pl.ds(base, CH)], idx_v)
                iv = idx_v[...]
                pltpu.async_copy(feat_hbm.at[iv], fbuf, sem0).wait()

                def edge_body(i, den_i):
                    valid = jnp.logical_and((base + i) >= s0, (base + i) < t0)
                    arow = fbuf[i, pl.ds(feat_dim, 16)]
                    sv = arow + dvec
                    sv = jnp.where(sv > 0, sv, 0.2 * sv)
                    ev = jnp.exp(sv)
                    validf = jnp.where(valid, 1.0, 0.0)
                    ev = ev * jnp.where(lanemask, validf, 0.0)
                    xr = [fbuf[i, pl.ds(j * 16, 16)] for j in range(nj)]
                    for k in range(n_heads):
                        skv = _lane_bcast(ev, k)
                        for j in range(nj):
                            acc[pl.ds(k * feat_dim + j * 16, 16)] += skv * xr[j]
                    return den_i + ev

                return lax.fori_loop(0, CH, edge_body, den)

            return chunk_body

        def group_body(g, _):
            va = rp_v[pl.ds(g * 16, 16)]
            vb = rp_v[pl.ds(g * 16 + 16, 16)]
            for ni in range(16):
                s0 = va[ni]
                t0 = vb[0] if ni == 15 else va[ni + 1]
                nglob = g * 16 + ni
                dvec = nd_a[pl.ds(nglob * 16, 16)]

                def zero_body(j, _z):
                    acc[pl.ds(j * 16, 16)] = jnp.zeros((16,), jnp.float32)
                    return 0
                lax.fori_loop(0, out_w // 16, zero_body, 0)

                c0 = (s0 // CH) * CH
                nchunks = (t0 - c0 + CH - 1) // CH
                den = lax.fori_loop(0, nchunks, chunk_maker(s0, t0, dvec),
                                    jnp.zeros((16,), jnp.float32))
                for k in range(n_heads):
                    invv = 1.0 / _lane_bcast(den, k)

                    def norm_body(j, _z, k=k, invv=invv):
                        sl = pl.ds(k * feat_dim + j * 16, 16)
                        outrow[sl] = acc[sl] * invv
                        return 0
                    lax.fori_loop(0, nj, norm_body, 0)
                pltpu.sync_copy(outrow, out_hbm.at[lo + nglob])
            return 0

        lax.fori_loop(0, ROWS_PER // 16, group_body, 0)

    return agg


_agg1 = _make_agg(HEADS, IN_DIM)
_agg2 = _make_agg(1, HID)


def _mid_body(agg_ref, w1_ref, b1_ref, w2_ref, as2_ref, ad2_ref,
              f2_ref, d2_ref):
    blk = agg_ref.shape[0]
    g = jnp.zeros((blk, HID), jnp.float32)
    for k in range(HEADS):
        hk = jnp.dot(agg_ref[:, k * HID:(k + 1) * HID],
                     w1_ref[:, k * HID:(k + 1) * HID],
                     preferred_element_type=jnp.float32)
        hk = hk + b1_ref[0, k * HID:(k + 1) * HID][None, :]
        hk = jnp.where(hk > 0, hk, jnp.exp(hk) - 1.0)
        g = g + jnp.dot(hk, w2_ref[k * HID:(k + 1) * HID, :],
                        preferred_element_type=jnp.float32)
    a_s = jnp.dot(g, as2_ref[...].reshape(HID, 1),
                  preferred_element_type=jnp.float32)
    a_d = jnp.dot(g, ad2_ref[...].reshape(HID, 1),
                  preferred_element_type=jnp.float32)
    f2_ref[...] = jnp.concatenate(
        [g, a_s, jnp.zeros((blk, TW - HID - 1), jnp.float32)], axis=1)
    d2_ref[...] = jnp.concatenate(
        [a_d, jnp.zeros((blk, 15), jnp.float32)], axis=1)


def _mid(agg1, W1, b1, W2, att_src2, att_dst2):
    blk = 1000
    return pl.pallas_call(
        _mid_body,
        grid=(N // blk,),
        in_specs=[
            pl.BlockSpec((blk, HEADS * HID), lambda i: (i, 0)),
            pl.BlockSpec((IN_DIM, HEADS * HID), lambda i: (0, 0)),
            pl.BlockSpec((1, HEADS * HID), lambda i: (0, 0)),
            pl.BlockSpec((HEADS * HID, HID), lambda i: (0, 0)),
            pl.BlockSpec((1, 1, HID), lambda i: (0, 0, 0)),
            pl.BlockSpec((1, 1, HID), lambda i: (0, 0, 0)),
        ],
        out_specs=[
            pl.BlockSpec((blk, TW), lambda i: (i, 0)),
            pl.BlockSpec((blk, 16), lambda i: (i, 0)),
        ],
        out_shape=[
            jax.ShapeDtypeStruct((N, TW), jnp.float32),
            jax.ShapeDtypeStruct((N, 16), jnp.float32),
        ],
    )(agg1, W1, b1.reshape(1, -1), W2, att_src2, att_dst2)


def _head_body(agg2_ref, b2_ref, dw1_ref, db1_ref, dw2_ref, db2_ref, ct_ref,
               fused_ref, xr_ref, q_ref):
    fused = agg2_ref[...] + b2_ref[0, :][None, :]
    fused_ref[...] = fused
    z = jnp.dot(fused, dw1_ref[...], preferred_element_type=jnp.float32)
    z = z + db1_ref[0, :][None, :]
    z = jnp.where(z > 0, z, jnp.exp(z) - 1.0)
    xr_ref[...] = (jnp.dot(z, dw2_ref[...], preferred_element_type=jnp.float32)
                   + db2_ref[0, :][None, :])
    ct = ct_ref[...]
    f2 = jnp.sum(fused * fused, axis=1, keepdims=True)
    c2 = jnp.sum(ct * ct, axis=0, keepdims=True)
    dist = f2 - 2.0 * jnp.dot(fused, ct, preferred_element_type=jnp.float32) + c2
    qv = 1.0 / (1.0 + dist)
    qv = qv - jnp.max(qv, axis=1, keepdims=True)
    eq = jnp.exp(qv)
    q_ref[...] = eq / jnp.sum(eq, axis=1, keepdims=True)


def _head(agg2, b2, dec_W1, dec_b1, dec_W2, dec_b2, centersT):
    blk = 1000
    return pl.pallas_call(
        _head_body,
        grid=(N // blk,),
        in_specs=[
            pl.BlockSpec((blk, HID), lambda i: (i, 0)),
            pl.BlockSpec((1, HID), lambda i: (0, 0)),
            pl.BlockSpec((HID, 512), lambda i: (0, 0)),
            pl.BlockSpec((1, 512), lambda i: (0, 0)),
            pl.BlockSpec((512, IN_DIM), lambda i: (0, 0)),
            pl.BlockSpec((1, IN_DIM), lambda i: (0, 0)),
            pl.BlockSpec((HID, NUM_CLASSES), lambda i: (0, 0)),
        ],
        out_specs=[
            pl.BlockSpec((blk, HID), lambda i: (i, 0)),
            pl.BlockSpec((blk, IN_DIM), lambda i: (i, 0)),
            pl.BlockSpec((blk, NUM_CLASSES), lambda i: (i, 0)),
        ],
        out_shape=[
            jax.ShapeDtypeStruct((N, HID), jnp.float32),
            jax.ShapeDtypeStruct((N, IN_DIM), jnp.float32),
            jax.ShapeDtypeStruct((N, NUM_CLASSES), jnp.float32),
        ],
    )(agg2, b2.reshape(1, -1), dec_W1, dec_b1.reshape(1, -1),
      dec_W2, dec_b2.reshape(1, -1), centersT)


def kernel(x, edge_index, W1, att_src1, att_dst1, b1, W2, att_src2, att_dst2,
           b2, dec_W1, dec_b1, dec_W2, dec_b2, centers):
    loop = jnp.arange(N, dtype=edge_index.dtype)
    src = jnp.concatenate([edge_index[0], loop])
    dst = jnp.concatenate([edge_index[1], loop])
    EB = 1024
    NB = (EP + EB - 1) // EB
    EPAD = NB * EB
    SENT = N
    dstp = jnp.concatenate(
        [dst.astype(jnp.int32), jnp.full((EPAD - EP,), SENT, jnp.int32)])
    srcp = jnp.concatenate(
        [src.astype(jnp.int32), jnp.zeros((EPAD - EP,), jnp.int32)])
    dstb = dstp.reshape(NB, EB)
    ii = lax.broadcasted_iota(jnp.int32, (EB, EB), 0)
    jj = lax.broadcasted_iota(jnp.int32, (EB, EB), 1)
    eq = dstb[:, :, None] == dstb[:, None, :]
    occ_w = jnp.sum(jnp.where(jnp.logical_and(eq, (jj < ii)[None]), 1, 0),
                    axis=2).reshape(-1)
    nodes = jnp.arange(N + 1, dtype=jnp.int32)
    hist = jnp.sum((dstb[:, :, None] == nodes[None, None, :]).astype(jnp.int32),
                   axis=1)
    cross = jnp.concatenate(
        [jnp.zeros((1, N + 1), jnp.int32),
         jnp.cumsum(hist, axis=0)[:-1]], axis=0)
    tot = cross[-1] + hist[-1]
    row_ptr_all = jnp.concatenate(
        [jnp.zeros((1,), jnp.int32), jnp.cumsum(tot)])
    bidx = jnp.arange(EPAD, dtype=jnp.int32) // EB
    dest = row_ptr_all[dstp] + cross.reshape(-1)[bidx * (N + 1) + dstp] + occ_w
    src_s = jnp.zeros((EPAD,), jnp.int32).at[dest].set(
        srcp, mode="promise_in_bounds", unique_indices=True)
    row_ptr = row_ptr_all[:N + 1]
    rp_pad = jnp.concatenate(
        [row_ptr, jnp.full((RP_PAD - (N + 1),), EP, jnp.int32)])
    src_pad = jnp.concatenate(
        [src_s[:EP], jnp.zeros((SRC_PAD - EP,), jnp.int32)])

    f1, d1 = _score1(x, W1, att_src1, att_dst1)
    d1_pad = jnp.concatenate(
        [d1, jnp.zeros((NPAD - N, 16), jnp.float32)], axis=0).reshape(-1)
    agg1 = _agg1(f1, d1_pad, src_pad, rp_pad)[:N]

    f2, d2 = _mid(agg1, W1, b1, W2, att_src2, att_dst2)
    d2_pad = jnp.concatenate(
        [d2, jnp.zeros((NPAD - N, 16), jnp.float32)], axis=0).reshape(-1)
    agg2 = _agg2(f2, d2_pad, src_pad, rp_pad)[:N]

    fused, x_recon, q = _head(agg2, b2, dec_W1, dec_b1, dec_W2, dec_b2,
                              centers.T)
    return (fused, x_recon, q)


# speedup vs baseline: 1.6851x; 1.1570x over previous
import functools

import jax
import jax.numpy as jnp
from jax import lax
from jax.experimental import pallas as pl
from jax.experimental.pallas import tpu as pltpu
from jax.experimental.pallas import tpu_sc as plsc

N = 10000
E = 160000
EP = E + N
IN_DIM = 256
HID = 256
HEADS = 8
NUM_CLASSES = 7

NW = 32
ROWS_PER = 320
NPAD = NW * ROWS_PER
RP_PAD = NPAD + 336
SRC_PAD = EP + 24
CH = 16


TW = 384


def _score1_body(x_ref, w1_ref, as_ref, ad_ref, f_ref, d_ref):
    w1 = w1_ref[...]
    asv = as_ref[...].reshape(1, HEADS * HID)
    adv = ad_ref[...].reshape(1, HEADS * HID)
    sel = (lax.broadcasted_iota(jnp.int32, (HEADS * HID, HEADS), 0) // HID
           == lax.broadcasted_iota(jnp.int32, (HEADS * HID, HEADS), 1)
           ).astype(jnp.float32)
    vs = jnp.dot(w1 * asv, sel, preferred_element_type=jnp.float32)
    vd = jnp.dot(w1 * adv, sel, preferred_element_type=jnp.float32)
    xb = x_ref[...]
    a_s = jnp.dot(xb, vs, preferred_element_type=jnp.float32)
    a_d = jnp.dot(xb, vd, preferred_element_type=jnp.float32)
    blk = xb.shape[0]
    f_ref[...] = jnp.concatenate(
        [xb, a_s, jnp.zeros((blk, TW - IN_DIM - HEADS), jnp.float32)], axis=1)
    d_ref[...] = jnp.concatenate(
        [a_d, jnp.zeros((blk, 8), jnp.float32)], axis=1)


def _score1(x, W1, att_src1, att_dst1):
    blk = 2000
    return pl.pallas_call(
        _score1_body,
        grid=(N // blk,),
        in_specs=[
            pl.BlockSpec((blk, IN_DIM), lambda i: (i, 0)),
            pl.BlockSpec((IN_DIM, HEADS * HID), lambda i: (0, 0)),
            pl.BlockSpec((1, HEADS, HID), lambda i: (0, 0, 0)),
            pl.BlockSpec((1, HEADS, HID), lambda i: (0, 0, 0)),
        ],
        out_specs=[
            pl.BlockSpec((blk, TW), lambda i: (i, 0)),
            pl.BlockSpec((blk, 16), lambda i: (i, 0)),
        ],
        out_shape=[
            jax.ShapeDtypeStruct((N, TW), jnp.float32),
            jax.ShapeDtypeStruct((N, 16), jnp.float32),
        ],
    )(x, W1, att_src1, att_dst1)


def _make_agg(n_heads, feat_dim):
    out_w = n_heads * feat_dim
    nj = feat_dim // 16
    mesh = plsc.VectorSubcoreMesh(core_axis_name="c", subcore_axis_name="s")

    def _lane_bcast(vec, k):
        idx = jnp.full((16, 1), k, jnp.int32)
        dn = lax.GatherDimensionNumbers(offset_dims=(),
                                        collapsed_slice_dims=(0,),
                                        start_index_map=(0,))
        return lax.gather(vec, idx, dn, slice_sizes=(1,),
                          mode=lax.GatherScatterMode.PROMISE_IN_BOUNDS)

    @functools.partial(
        pl.kernel,
        out_type=jax.ShapeDtypeStruct((NPAD, out_w), jnp.float32),
        mesh=mesh,
        scratch_types=[
            pltpu.VMEM((336,), jnp.int32),
            pltpu.VMEM((ROWS_PER * 16,), jnp.float32),
            pltpu.VMEM((16,), jnp.int32),
            pltpu.VMEM((CH, TW), jnp.float32),
            pltpu.VMEM((out_w,), jnp.float32),
            pltpu.VMEM((out_w,), jnp.float32),
            pltpu.SemaphoreType.DMA,
        ],
    )
    def agg(feat_hbm, dtab_hbm, src_hbm, rp_hbm, out_hbm,
            rp_v, nd_a, idx_v, fbuf, acc, outrow, sem0):
        w = lax.axis_index("s") * 2 + lax.axis_index("c")
        lo = w * ROWS_PER
        pltpu.sync_copy(rp_hbm.at[pl.ds(lo, 328)], rp_v.at[pl.ds(0, 328)])
        pltpu.sync_copy(dtab_hbm.at[pl.ds(lo * 16, ROWS_PER * 16)], nd_a)
        lanemask = lax.iota(jnp.int32, 16) < n_heads

        def chunk_maker(s0, t0, dvec):
            def chunk_body(ci, den):
                base = (s0 // CH) * CH + ci * CH
                pltpu.sync_copy(src_hbm.at[pl.ds(base, CH)], idx_v)
                iv = idx_v[...]
                pltpu.async_copy(feat_hbm.at[iv], fbuf, sem0).wait()

                def edge_body(i, den_i):
                    valid = jnp.logical_and((base + i) >= s0, (base + i) < t0)
                    arow = fbuf[i, pl.ds(feat_dim, 16)]
                    sv = arow + dvec
                    sv = jnp.where(sv > 0, sv, 0.2 * sv)
                    ev = jnp.exp(sv)
                    validf = jnp.where(valid, 1.0, 0.0)
                    ev = ev * jnp.where(lanemask, validf, 0.0)
                    xr = [fbuf[i, pl.ds(j * 16, 16)] for j in range(nj)]
                    for k in range(n_heads):
                        skv = _lane_bcast(ev, k)
                        for j in range(nj):
                            acc[pl.ds(k * feat_dim + j * 16, 16)] += skv * xr[j]
                    return den_i + ev

                return lax.fori_loop(0, CH, edge_body, den)

            return chunk_body

        def group_body(g, _):
            va = rp_v[pl.ds(g * 16, 16)]
            vb = rp_v[pl.ds(g * 16 + 16, 16)]
            for ni in range(16):
                s0 = va[ni]
                t0 = vb[0] if ni == 15 else va[ni + 1]
                nglob = g * 16 + ni
                dvec = nd_a[pl.ds(nglob * 16, 16)]

                def zero_body(j, _z):
                    acc[pl.ds(j * 16, 16)] = jnp.zeros((16,), jnp.float32)
                    return 0
                lax.fori_loop(0, out_w // 16, zero_body, 0)

                c0 = (s0 // CH) * CH
                nchunks = (t0 - c0 + CH - 1) // CH
                den = lax.fori_loop(0, nchunks, chunk_maker(s0, t0, dvec),
                                    jnp.zeros((16,), jnp.float32))
                for k in range(n_heads):
                    invv = 1.0 / _lane_bcast(den, k)

                    def norm_body(j, _z, k=k, invv=invv):
                        sl = pl.ds(k * feat_dim + j * 16, 16)
                        outrow[sl] = acc[sl] * invv
                        return 0
                    lax.fori_loop(0, nj, norm_body, 0)
                pltpu.sync_copy(outrow, out_hbm.at[lo + nglob])
            return 0

        lax.fori_loop(0, ROWS_PER // 16, group_body, 0)

    return agg


_agg1 = _make_agg(HEADS, IN_DIM)
_agg2 = _make_agg(1, HID)


def _mid_body(agg_ref, w1_ref, b1_ref, w2_ref, as2_ref, ad2_ref,
              f2_ref, d2_ref):
    blk = agg_ref.shape[0]
    g = jnp.zeros((blk, HID), jnp.float32)
    for k in range(HEADS):
        hk = jnp.dot(agg_ref[:, k * HID:(k + 1) * HID],
                     w1_ref[:, k * HID:(k + 1) * HID],
                     preferred_element_type=jnp.float32)
        hk = hk + b1_ref[0, k * HID:(k + 1) * HID][None, :]
        hk = jnp.where(hk > 0, hk, jnp.exp(hk) - 1.0)
        g = g + jnp.dot(hk, w2_ref[k * HID:(k + 1) * HID, :],
                        preferred_element_type=jnp.float32)
    a_s = jnp.dot(g, as2_ref[...].reshape(HID, 1),
                  preferred_element_type=jnp.float32)
    a_d = jnp.dot(g, ad2_ref[...].reshape(HID, 1),
                  preferred_element_type=jnp.float32)
    f2_ref[...] = jnp.concatenate(
        [g, a_s, jnp.zeros((blk, TW - HID - 1), jnp.float32)], axis=1)
    d2_ref[...] = jnp.concatenate(
        [a_d, jnp.zeros((blk, 15), jnp.float32)], axis=1)


def _mid(agg1, W1, b1, W2, att_src2, att_dst2):
    blk = 1000
    return pl.pallas_call(
        _mid_body,
        grid=(N // blk,),
        in_specs=[
            pl.BlockSpec((blk, HEADS * HID), lambda i: (i, 0)),
            pl.BlockSpec((IN_DIM, HEADS * HID), lambda i: (0, 0)),
            pl.BlockSpec((1, HEADS * HID), lambda i: (0, 0)),
            pl.BlockSpec((HEADS * HID, HID), lambda i: (0, 0)),
            pl.BlockSpec((1, 1, HID), lambda i: (0, 0, 0)),
            pl.BlockSpec((1, 1, HID), lambda i: (0, 0, 0)),
        ],
        out_specs=[
            pl.BlockSpec((blk, TW), lambda i: (i, 0)),
            pl.BlockSpec((blk, 16), lambda i: (i, 0)),
        ],
        out_shape=[
            jax.ShapeDtypeStruct((N, TW), jnp.float32),
            jax.ShapeDtypeStruct((N, 16), jnp.float32),
        ],
    )(agg1, W1, b1.reshape(1, -1), W2, att_src2, att_dst2)


def _head_body(agg2_ref, b2_ref, dw1_ref, db1_ref, dw2_ref, db2_ref, ct_ref,
               fused_ref, xr_ref, q_ref):
    fused = agg2_ref[...] + b2_ref[0, :][None, :]
    fused_ref[...] = fused
    z = jnp.dot(fused, dw1_ref[...], preferred_element_type=jnp.float32)
    z = z + db1_ref[0, :][None, :]
    z = jnp.where(z > 0, z, jnp.exp(z) - 1.0)
    xr_ref[...] = (jnp.dot(z, dw2_ref[...], preferred_element_type=jnp.float32)
                   + db2_ref[0, :][None, :])
    ct = ct_ref[...]
    f2 = jnp.sum(fused * fused, axis=1, keepdims=True)
    c2 = jnp.sum(ct * ct, axis=0, keepdims=True)
    dist = f2 - 2.0 * jnp.dot(fused, ct, preferred_element_type=jnp.float32) + c2
    qv = 1.0 / (1.0 + dist)
    qv = qv - jnp.max(qv, axis=1, keepdims=True)
    eq = jnp.exp(qv)
    q_ref[...] = eq / jnp.sum(eq, axis=1, keepdims=True)


def _head(agg2, b2, dec_W1, dec_b1, dec_W2, dec_b2, centersT):
    blk = 1000
    return pl.pallas_call(
        _head_body,
        grid=(N // blk,),
        in_specs=[
            pl.BlockSpec((blk, HID), lambda i: (i, 0)),
            pl.BlockSpec((1, HID), lambda i: (0, 0)),
            pl.BlockSpec((HID, 512), lambda i: (0, 0)),
            pl.BlockSpec((1, 512), lambda i: (0, 0)),
            pl.BlockSpec((512, IN_DIM), lambda i: (0, 0)),
            pl.BlockSpec((1, IN_DIM), lambda i: (0, 0)),
            pl.BlockSpec((HID, NUM_CLASSES), lambda i: (0, 0)),
        ],
        out_specs=[
            pl.BlockSpec((blk, HID), lambda i: (i, 0)),
            pl.BlockSpec((blk, IN_DIM), lambda i: (i, 0)),
            pl.BlockSpec((blk, NUM_CLASSES), lambda i: (i, 0)),
        ],
        out_shape=[
            jax.ShapeDtypeStruct((N, HID), jnp.float32),
            jax.ShapeDtypeStruct((N, IN_DIM), jnp.float32),
            jax.ShapeDtypeStruct((N, NUM_CLASSES), jnp.float32),
        ],
    )(agg2, b2.reshape(1, -1), dec_W1, dec_b1.reshape(1, -1),
      dec_W2, dec_b2.reshape(1, -1), centersT)


def kernel(x, edge_index, W1, att_src1, att_dst1, b1, W2, att_src2, att_dst2,
           b2, dec_W1, dec_b1, dec_W2, dec_b2, centers):
    loop = jnp.arange(N, dtype=edge_index.dtype)
    src = jnp.concatenate([edge_index[0], loop])
    dst = jnp.concatenate([edge_index[1], loop])
    EB = 1024
    NB = (EP + EB - 1) // EB
    EPAD = NB * EB
    SENT = N
    dstp = jnp.concatenate(
        [dst.astype(jnp.int32), jnp.full((EPAD - EP,), SENT, jnp.int32)])
    srcp = jnp.concatenate(
        [src.astype(jnp.int32), jnp.zeros((EPAD - EP,), jnp.int32)])
    dstb = dstp.reshape(NB, EB)
    ii = lax.broadcasted_iota(jnp.int32, (EB, EB), 0)
    jj = lax.broadcasted_iota(jnp.int32, (EB, EB), 1)
    eq = dstb[:, :, None] == dstb[:, None, :]
    occ_w = jnp.sum(jnp.where(jnp.logical_and(eq, (jj < ii)[None]), 1, 0),
                    axis=2).reshape(-1)
    HI, LO = 80, 128
    hi = dstb // LO
    lo_d = dstb % LO
    ohhi = (hi[:, :, None]
            == jnp.arange(HI, dtype=jnp.int32)[None, None, :]
            ).astype(jnp.bfloat16)
    ohlo = (lo_d[:, :, None]
            == jnp.arange(LO, dtype=jnp.int32)[None, None, :]
            ).astype(jnp.bfloat16)
    hist = jnp.einsum("bih,bil->bhl", ohhi, ohlo,
                      preferred_element_type=jnp.float32
                      ).reshape(NB, HI * LO)
    cross = jnp.concatenate(
        [jnp.zeros((1, HI * LO), jnp.float32),
         jnp.cumsum(hist, axis=0)[:-1]], axis=0)
    tot = cross[-1] + hist[-1]
    row_ptr_all = jnp.concatenate(
        [jnp.zeros((1,), jnp.float32), jnp.cumsum(tot)]).astype(jnp.int32)
    bidx = jnp.arange(EPAD, dtype=jnp.int32) // EB
    dest = (row_ptr_all[dstp]
            + cross.reshape(-1)[bidx * (HI * LO) + dstp].astype(jnp.int32)
            + occ_w)
    src_s = jnp.zeros((EPAD,), jnp.int32).at[dest].set(
        srcp, mode="promise_in_bounds", unique_indices=True)
    row_ptr = row_ptr_all[:N + 1]
    rp_pad = jnp.concatenate(
        [row_ptr, jnp.full((RP_PAD - (N + 1),), EP, jnp.int32)])
    src_pad = jnp.concatenate(
        [src_s[:EP], jnp.zeros((SRC_PAD - EP,), jnp.int32)])

    f1, d1 = _score1(x, W1, att_src1, att_dst1)
    d1_pad = jnp.concatenate(
        [d1, jnp.zeros((NPAD - N, 16), jnp.float32)], axis=0).reshape(-1)
    agg1 = _agg1(f1, d1_pad, src_pad, rp_pad)[:N]

    f2, d2 = _mid(agg1, W1, b1, W2, att_src2, att_dst2)
    d2_pad = jnp.concatenate(
        [d2, jnp.zeros((NPAD - N, 16), jnp.float32)], axis=0).reshape(-1)
    agg2 = _agg2(f2, d2_pad, src_pad, rp_pad)[:N]

    fused, x_recon, q = _head(agg2, b2, dec_W1, dec_b1, dec_W2, dec_b2,
                              centers.T)
    return (fused, x_recon, q)
